# Initial kernel scaffold; baseline (speedup 1.0000x reference)
import jax
import jax.numpy as jnp
from jax.experimental import pallas as pl


def kernel(embed, bias, W_enc, lookup):
    raise NotImplementedError("write your pallas kernel here")


# speedup vs baseline: 1.2086x; 1.2086x over previous
import jax
import jax.numpy as jnp
from jax.experimental import pallas as pl

TOPK = 32


def _copy_body(x_ref, o_ref):
    o_ref[...] = x_ref[...]


def kernel(embed, bias, W_enc, lookup):
    embed0 = embed - bias
    project = jax.lax.dot_general(
        embed0.astype(jnp.bfloat16), W_enc.astype(jnp.bfloat16),
        (((1,), (1,)), ((), ())),
        preferred_element_type=jnp.float32)
    weights, feats = jax.lax.top_k(project, TOPK)
    vecs = jnp.take(lookup, feats, axis=0)
    out = jnp.einsum('bte,bt->be', vecs, weights,
                     precision=jax.lax.Precision.HIGHEST) + bias
    return pl.pallas_call(
        _copy_body,
        grid=(8,),
        in_specs=[pl.BlockSpec((512, 2048), lambda i: (i, 0))],
        out_specs=pl.BlockSpec((512, 2048), lambda i: (i, 0)),
        out_shape=jax.ShapeDtypeStruct(out.shape, out.dtype),
    )(out)


# speedup vs baseline: 3.0641x; 2.5352x over previous
import functools

import jax
import jax.numpy as jnp
from jax import lax
from jax.experimental import pallas as pl
from jax.experimental.pallas import tpu as pltpu
from jax.experimental.pallas import tpu_sc as plsc

B = 4096
EMBED = 2048
F = 16384
K = 32
L = 16


BM = 512
BN = 1024


def _mm_body(x_ref, b_ref, w_ref, o_ref):
    x = x_ref[...] - b_ref[...]
    o_ref[...] = lax.dot_general(
        x, w_ref[...], (((1,), (1,)), ((), ())),
        preferred_element_type=jnp.float32)


def _encode(embed, bias2d, W_enc):
    return pl.pallas_call(
        _mm_body,
        grid=(B // BM, F // BN),
        in_specs=[
            pl.BlockSpec((BM, EMBED), lambda i, j: (i, 0)),
            pl.BlockSpec((1, EMBED), lambda i, j: (0, 0)),
            pl.BlockSpec((BN, EMBED), lambda i, j: (j, 0)),
        ],
        out_specs=pl.BlockSpec((BM, BN), lambda i, j: (i, j)),
        out_shape=jax.ShapeDtypeStruct((B, F), jnp.float32),
    )(embed, bias2d, W_enc)


TM = 128


def _topk_body(p_ref, w_ref, f_ref):
    x = p_ref[...]
    iota = lax.broadcasted_iota(jnp.int32, (TM, F), 1)
    wcols = []
    fcols = []
    for _ in range(K):
        m = jnp.max(x, axis=1, keepdims=True)
        idx = jnp.min(jnp.where(x >= m, iota, F), axis=1, keepdims=True)
        wcols.append(m)
        fcols.append(idx)
        x = jnp.where(iota == idx, -jnp.inf, x)
    w_ref[...] = jnp.concatenate(wcols, axis=1)
    f_ref[...] = jnp.concatenate(fcols, axis=1)


def _topk(project):
    return pl.pallas_call(
        _topk_body,
        grid=(B // TM,),
        in_specs=[pl.BlockSpec((TM, F), lambda i: (i, 0))],
        out_specs=[
            pl.BlockSpec((TM, K), lambda i: (i, 0)),
            pl.BlockSpec((TM, K), lambda i: (i, 0)),
        ],
        out_shape=[
            jax.ShapeDtypeStruct((B, K), jnp.float32),
            jax.ShapeDtypeStruct((B, K), jnp.int32),
        ],
    )(project)


NC = 2
NS = 16
NW = NC * NS
ROWS_PER_W = B // NW


def _decode_body(lookup_hbm, feats_hbm, wexp_hbm, bias_hbm, out_hbm,
                 idx_v, wexp_v, rows_v, bias_v, out_v, sem):
    wid = lax.axis_index("s") * NC + lax.axis_index("c")
    pltpu.sync_copy(bias_hbm, bias_v)

    def row_body(r, carry):
        row = wid * ROWS_PER_W + r
        pltpu.sync_copy(feats_hbm.at[row], idx_v)
        pltpu.sync_copy(wexp_hbm.at[row], wexp_v)
        pltpu.async_copy(lookup_hbm.at[idx_v], rows_v, sem).wait()

        def chunk_body(c, _):
            off = pl.multiple_of(c * L, L)
            acc = bias_v[pl.ds(off, L)]
            for t in range(K):
                acc = acc + wexp_v[t, :] * rows_v[t, pl.ds(off, L)]
            out_v[pl.ds(off, L)] = acc
            return 0

        lax.fori_loop(0, EMBED // L, chunk_body, 0)
        pltpu.sync_copy(out_v, out_hbm.at[row])
        return carry

    lax.fori_loop(0, ROWS_PER_W, row_body, 0)


def _decode(lookup, feats, wexp, bias):
    mesh = plsc.VectorSubcoreMesh(core_axis_name="c", subcore_axis_name="s")
    fn = functools.partial(
        pl.kernel,
        mesh=mesh,
        out_type=jax.ShapeDtypeStruct((B, EMBED), jnp.float32),
        scratch_types=[
            pltpu.VMEM((K,), jnp.int32),
            pltpu.VMEM((K, L), jnp.float32),
            pltpu.VMEM((K, EMBED), jnp.float32),
            pltpu.VMEM((EMBED,), jnp.float32),
            pltpu.VMEM((EMBED,), jnp.float32),
            pltpu.SemaphoreType.DMA,
        ],
    )(_decode_body)
    return fn(lookup, feats, wexp, bias)


def kernel(embed, bias, W_enc, lookup):
    project = _encode(embed, bias.reshape(1, EMBED), W_enc)
    weights, feats = _topk(project)
    wexp = jnp.broadcast_to(weights[:, :, None], (B, K, L)) + jnp.zeros(
        (B, K, L), jnp.float32)
    return _decode(lookup, feats, wexp, bias)


# speedup vs baseline: 3.3144x; 1.0817x over previous
import functools

import jax
import jax.numpy as jnp
from jax import lax
from jax.experimental import pallas as pl
from jax.experimental.pallas import tpu as pltpu
from jax.experimental.pallas import tpu_sc as plsc

B = 4096
EMBED = 2048
F = 16384
K = 32
L = 16


BM = 512
BN = 1024


def _mm_body(x_ref, b_ref, w_ref, o_ref):
    x = x_ref[...] - b_ref[...]
    o_ref[...] = lax.dot_general(
        x, w_ref[...], (((1,), (1,)), ((), ())),
        preferred_element_type=jnp.float32)


def _encode(embed, bias2d, W_enc):
    return pl.pallas_call(
        _mm_body,
        grid=(B // BM, F // BN),
        in_specs=[
            pl.BlockSpec((BM, EMBED), lambda i, j: (i, 0)),
            pl.BlockSpec((1, EMBED), lambda i, j: (0, 0)),
            pl.BlockSpec((BN, EMBED), lambda i, j: (j, 0)),
        ],
        out_specs=pl.BlockSpec((BM, BN), lambda i, j: (i, j)),
        out_shape=jax.ShapeDtypeStruct((B, F), jnp.float32),
    )(embed, bias2d, W_enc)


TM = 128


def _topk_body(p_ref, w_ref, f_ref):
    x = p_ref[...]
    iota = lax.broadcasted_iota(jnp.int32, (TM, F), 1)
    wcols = []
    fcols = []
    for _ in range(K):
        m = jnp.max(x, axis=1, keepdims=True)
        idx = jnp.min(jnp.where(x >= m, iota, F), axis=1, keepdims=True)
        wcols.append(m)
        fcols.append(idx)
        x = jnp.where(iota == idx, -jnp.inf, x)
    w_ref[...] = jnp.concatenate(wcols, axis=1)
    f_ref[...] = jnp.concatenate(fcols, axis=1)


def _topk(project):
    return pl.pallas_call(
        _topk_body,
        grid=(B // TM,),
        in_specs=[pl.BlockSpec((TM, F), lambda i: (i, 0))],
        out_specs=[
            pl.BlockSpec((TM, K), lambda i: (i, 0)),
            pl.BlockSpec((TM, K), lambda i: (i, 0)),
        ],
        out_shape=[
            jax.ShapeDtypeStruct((B, K), jnp.float32),
            jax.ShapeDtypeStruct((B, K), jnp.int32),
        ],
    )(project)


NC = 2
NS = 16
NW = NC * NS
ROWS_PER_W = B // NW


def _decode_body(lookup_hbm, feats_hbm, wexp_hbm, bias_hbm, out_hbm,
                 idx_all, wexp_v, bufA, bufB, bias_v, out_v, semA, semB):
    wid = lax.axis_index("s") * NC + lax.axis_index("c")
    base = wid * ROWS_PER_W
    pltpu.sync_copy(bias_hbm, bias_v)
    pltpu.sync_copy(feats_hbm.at[pl.ds(base, ROWS_PER_W)], idx_all)
    bufs = (bufA, bufB)
    sems = (semA, semB)
    nchunk = ROWS_PER_W * 2

    def issue(c, b):
        r = lax.div(c, 2)
        h = lax.rem(c, 2)
        pltpu.async_copy(
            lookup_hbm.at[idx_all.at[r, pl.ds(h * L, L)]], bufs[b], sems[b])

    def wait(b):
        pltpu.make_async_copy(
            lookup_hbm.at[pl.ds(0, L)], bufs[b], sems[b]).wait()

    issue(0, 0)

    def row_body(r, carry):
        issue(2 * r + 1, 1)
        wait(0)
        pltpu.sync_copy(wexp_hbm.at[base + r], wexp_v)

        def cb0(c, _):
            off = pl.multiple_of(c * L, L)
            acc = bias_v[pl.ds(off, L)]
            for t in range(L):
                acc = acc + wexp_v[t, :] * bufA[t, pl.ds(off, L)]
            out_v[pl.ds(off, L)] = acc
            return 0

        lax.fori_loop(0, EMBED // L, cb0, 0)

        @pl.when(2 * r + 2 < nchunk)
        def _():
            issue(2 * r + 2, 0)

        wait(1)

        def cb1(c, _):
            off = pl.multiple_of(c * L, L)
            acc = out_v[pl.ds(off, L)]
            for t in range(L):
                acc = acc + wexp_v[L + t, :] * bufB[t, pl.ds(off, L)]
            out_v[pl.ds(off, L)] = acc
            return 0

        lax.fori_loop(0, EMBED // L, cb1, 0)
        pltpu.sync_copy(out_v, out_hbm.at[base + r])
        return carry

    lax.fori_loop(0, ROWS_PER_W, row_body, 0)


def _decode(lookup, feats, wexp, bias):
    mesh = plsc.VectorSubcoreMesh(core_axis_name="c", subcore_axis_name="s")
    fn = functools.partial(
        pl.kernel,
        mesh=mesh,
        out_type=jax.ShapeDtypeStruct((B, EMBED), jnp.float32),
        scratch_types=[
            pltpu.VMEM((ROWS_PER_W, K), jnp.int32),
            pltpu.VMEM((K, L), jnp.float32),
            pltpu.VMEM((L, EMBED), jnp.float32),
            pltpu.VMEM((L, EMBED), jnp.float32),
            pltpu.VMEM((EMBED,), jnp.float32),
            pltpu.VMEM((EMBED,), jnp.float32),
            pltpu.SemaphoreType.DMA,
            pltpu.SemaphoreType.DMA,
        ],
    )(_decode_body)
    return fn(lookup, feats, wexp, bias)


def kernel(embed, bias, W_enc, lookup):
    project = _encode(embed, bias.reshape(1, EMBED), W_enc)
    weights, feats = _topk(project)
    wexp = jnp.broadcast_to(weights[:, :, None], (B, K, L)) + jnp.zeros(
        (B, K, L), jnp.float32)
    return _decode(lookup, feats, wexp, bias)


# speedup vs baseline: 3.4218x; 1.0324x over previous
import functools

import jax
import jax.numpy as jnp
from jax import lax
from jax.experimental import pallas as pl
from jax.experimental.pallas import tpu as pltpu
from jax.experimental.pallas import tpu_sc as plsc

B = 4096
EMBED = 2048
F = 16384
K = 32
L = 16


BM = 1024
BN = 1024


def _mm_body(x_ref, b_ref, w_ref, o_ref):
    x = x_ref[...] - b_ref[...]
    o_ref[...] = lax.dot_general(
        x, w_ref[...], (((1,), (1,)), ((), ())),
        preferred_element_type=jnp.float32)


def _encode(embed, bias2d, W_enc):
    return pl.pallas_call(
        _mm_body,
        grid=(B // BM, F // BN),
        in_specs=[
            pl.BlockSpec((BM, EMBED), lambda i, j: (i, 0)),
            pl.BlockSpec((1, EMBED), lambda i, j: (0, 0)),
            pl.BlockSpec((BN, EMBED), lambda i, j: (j, 0)),
        ],
        out_specs=pl.BlockSpec((BM, BN), lambda i, j: (i, j)),
        out_shape=jax.ShapeDtypeStruct((B, F), jnp.float32),
    )(embed, bias2d, W_enc)


TM = 128


def _topk_body(p_ref, w_ref, f_ref):
    x = p_ref[...]
    iota = lax.broadcasted_iota(jnp.int32, (TM, F), 1)
    wcols = []
    fcols = []
    for _ in range(K):
        m = jnp.max(x, axis=1, keepdims=True)
        idx = jnp.min(jnp.where(x >= m, iota, F), axis=1, keepdims=True)
        wcols.append(m)
        fcols.append(idx)
        x = jnp.where(iota == idx, -jnp.inf, x)
    w_ref[...] = jnp.concatenate(wcols, axis=1)
    f_ref[...] = jnp.concatenate(fcols, axis=1)


def _topk(project):
    return pl.pallas_call(
        _topk_body,
        grid=(B // TM,),
        in_specs=[pl.BlockSpec((TM, F), lambda i: (i, 0))],
        out_specs=[
            pl.BlockSpec((TM, K), lambda i: (i, 0)),
            pl.BlockSpec((TM, K), lambda i: (i, 0)),
        ],
        out_shape=[
            jax.ShapeDtypeStruct((B, K), jnp.float32),
            jax.ShapeDtypeStruct((B, K), jnp.int32),
        ],
    )(project)


NC = 2
NS = 16
NW = NC * NS
ROWS_PER_W = B // NW


def _decode_body(lookup_hbm, feats_hbm, wexp_hbm, bias_hbm, out_hbm,
                 idx_all, wexp_v, bufA, bufB, bias_v, out_v, semA, semB):
    wid = lax.axis_index("s") * NC + lax.axis_index("c")
    base = wid * ROWS_PER_W
    pltpu.sync_copy(bias_hbm, bias_v)
    pltpu.sync_copy(feats_hbm.at[pl.ds(base, ROWS_PER_W)], idx_all)
    bufs = (bufA, bufB)
    sems = (semA, semB)
    nchunk = ROWS_PER_W * 2

    def issue(c, b):
        r = lax.div(c, 2)
        h = lax.rem(c, 2)
        pltpu.async_copy(
            lookup_hbm.at[idx_all.at[r, pl.ds(h * L, L)]], bufs[b], sems[b])

    def wait(b):
        pltpu.make_async_copy(
            lookup_hbm.at[pl.ds(0, L)], bufs[b], sems[b]).wait()

    issue(0, 0)

    def row_body(r, carry):
        issue(2 * r + 1, 1)
        wait(0)
        pltpu.sync_copy(wexp_hbm.at[base + r], wexp_v)

        def cb0(c, _):
            for u in range(4):
                off = pl.multiple_of(c * (4 * L) + u * L, L)
                acc = bias_v[pl.ds(off, L)]
                for t in range(L):
                    acc = acc + wexp_v[t, :] * bufA[t, pl.ds(off, L)]
                out_v[pl.ds(off, L)] = acc
            return 0

        lax.fori_loop(0, EMBED // (4 * L), cb0, 0)

        @pl.when(2 * r + 2 < nchunk)
        def _():
            issue(2 * r + 2, 0)

        wait(1)

        def cb1(c, _):
            for u in range(4):
                off = pl.multiple_of(c * (4 * L) + u * L, L)
                acc = out_v[pl.ds(off, L)]
                for t in range(L):
                    acc = acc + wexp_v[L + t, :] * bufB[t, pl.ds(off, L)]
                out_v[pl.ds(off, L)] = acc
            return 0

        lax.fori_loop(0, EMBED // (4 * L), cb1, 0)
        pltpu.sync_copy(out_v, out_hbm.at[base + r])
        return carry

    lax.fori_loop(0, ROWS_PER_W, row_body, 0)


def _decode(lookup, feats, wexp, bias):
    mesh = plsc.VectorSubcoreMesh(core_axis_name="c", subcore_axis_name="s")
    fn = functools.partial(
        pl.kernel,
        mesh=mesh,
        out_type=jax.ShapeDtypeStruct((B, EMBED), jnp.float32),
        scratch_types=[
            pltpu.VMEM((ROWS_PER_W, K), jnp.int32),
            pltpu.VMEM((K, L), jnp.float32),
            pltpu.VMEM((L, EMBED), jnp.float32),
            pltpu.VMEM((L, EMBED), jnp.float32),
            pltpu.VMEM((EMBED,), jnp.float32),
            pltpu.VMEM((EMBED,), jnp.float32),
            pltpu.SemaphoreType.DMA,
            pltpu.SemaphoreType.DMA,
        ],
    )(_decode_body)
    return fn(lookup, feats, wexp, bias)


def kernel(embed, bias, W_enc, lookup):
    project = _encode(embed, bias.reshape(1, EMBED), W_enc)
    weights, feats = _topk(project)
    wexp = jnp.broadcast_to(weights[:, :, None], (B, K, L)) + jnp.zeros(
        (B, K, L), jnp.float32)
    return _decode(lookup, feats, wexp, bias)
